# Initial kernel scaffold; baseline (speedup 1.0000x reference)
import jax
import jax.numpy as jnp
from jax.experimental import pallas as pl


def kernel(hd_map, prediction):
    raise NotImplementedError("write your pallas kernel here")


# speedup vs baseline: 55.5039x; 55.5039x over previous
import jax
import jax.numpy as jnp
from jax import lax
from jax.experimental import pallas as pl

_K1 = 21.7
_K2 = 40.0
_LN2 = 0.6931471805599453
_H = 512
_W = 512
_N = 128
_SENT_LO = -1.0e4
_SENT_HI = 1.0e5
_EMPTY_THR = 1.0e6


def _fused_body(hd_ref, pred_ref, out_ref):
    hd = hd_ref[...]
    rowf = lax.broadcasted_iota(jnp.int32, (_H, _W), 0).astype(jnp.float32)

    def table(mask):
        fwd = jnp.where(mask, rowf, _SENT_LO)
        bwd = jnp.where(mask, rowf, _SENT_HI)
        k = 1
        for _ in range(9):
            top = jnp.full((k, _W), _SENT_LO, jnp.float32)
            fwd = jnp.maximum(fwd, jnp.concatenate([top, fwd[:_H - k, :]], axis=0))
            bot = jnp.full((k, _W), _SENT_HI, jnp.float32)
            bwd = jnp.minimum(bwd, jnp.concatenate([bwd[k:, :], bot], axis=0))
            k *= 2
        return jnp.minimum((rowf - fwd) ** 2, (bwd - rowf) ** 2)

    t_in = table(hd != 0.0)
    t_out = table(hd == 0.0)

    shifted = jnp.concatenate([jnp.zeros((1, _W), jnp.float32), hd[:_H - 1, :]], axis=0)
    rmax = jnp.maximum(hd, shifted)
    shiftc = jnp.concatenate([jnp.zeros((_H, 1), jnp.float32), rmax[:, :_W - 1]], axis=1)
    nb = jnp.maximum(rmax, shiftc)

    pred = pred_ref[...]
    p0 = pred[:, 0:1]
    p1 = pred[:, 1:2]
    p0f = p0.astype(jnp.float32)
    p1f = p1.astype(jnp.float32)

    colid = lax.broadcasted_iota(jnp.int32, (_N, _W), 1)
    onehot_r = (colid == p0).astype(jnp.float32)
    cmat = jnp.concatenate([t_out, t_in, nb], axis=1)
    rows = jnp.dot(onehot_r, cmat, preferred_element_type=jnp.float32)

    colf = colid.astype(jnp.float32)
    q = (colf - p1f) ** 2
    m2o = jnp.min(q + rows[:, 0:_W], axis=1, keepdims=True)
    m2i = jnp.min(q + rows[:, _W:2 * _W], axis=1, keepdims=True)
    onehot_c = (colid == p1).astype(jnp.float32)
    nbv = jnp.sum(rows[:, 2 * _W:3 * _W] * onehot_c, axis=1, keepdims=True)

    fb = p0f * p0f + p1f * p1f
    m2o = jnp.where(m2o > _EMPTY_THR, fb, m2o)
    m2i = jnp.where(m2i > _EMPTY_THR, fb, m2i)

    anyn = (nbv > 0.5) & (p0 >= 1) & (p1 >= 1)
    valid = (p0 >= 0) & (p0 <= _H) & (p1 >= 0) & (p1 <= _W)
    dist_o = jnp.sqrt(m2o)
    loss = jnp.where(anyn, jnp.exp(dist_o * (_LN2 / _K2)) - 1.0,
                     jnp.exp(-m2i * (1.0 / _K1)))
    loss = jnp.where(valid, loss, 0.0)
    out_ref[...] = jnp.sum(loss, keepdims=True).reshape(1, 1) * (1.0 / _N)


def kernel(hd_map, prediction):
    out = pl.pallas_call(
        _fused_body,
        out_shape=jax.ShapeDtypeStruct((1, 1), jnp.float32),
    )(hd_map, prediction)
    return out[0, 0]
